# baseline (speedup 1.0000x reference)
import jax
import jax.numpy as jnp
from jax import lax
from jax.experimental import pallas as pl
from jax.experimental.pallas import tpu as pltpu
from jax.experimental.pallas import tpu_sc as plsc

N = 10000
E = 320000
D = 128
H = 8
OUT = 8
F1 = 64
C = 16

NTILES = 32
EB = 128
NCHUNK = 80
EP = NTILES * NCHUNK * EB
PER_TILE = EP // NTILES
NP = 10112
ROWS_PER_SUB = NP // 16

_f32 = jnp.float32
_i32 = jnp.int32


def _node1_body(x_ref, w_ref, ad64_ref, as64_ref, ad16_ref, as16_ref,
                t_ref, td_ref, nmd_ref, nms_ref):
    i = pl.program_id(0)
    h = jnp.dot(x_ref[...], w_ref[...], preferred_element_type=_f32)
    ad64 = jnp.dot(h, ad64_ref[...], preferred_element_type=_f32)
    as64 = jnp.dot(h, as64_ref[...], preferred_element_type=_f32)
    ad16 = jnp.dot(h, ad16_ref[...], preferred_element_type=_f32)
    as16 = jnp.dot(h, as16_ref[...], preferred_element_type=_f32)
    t_ref[...] = jnp.concatenate([as64, h, as16], axis=1)
    td_ref[...] = jnp.concatenate([ad64, ad16], axis=1)
    md = jnp.broadcast_to(jnp.max(ad16, axis=0, keepdims=True), (8, 16))
    ms = jnp.broadcast_to(jnp.max(as16, axis=0, keepdims=True), (8, 16))

    @pl.when(i == 0)
    def _():
        nmd_ref[...] = md
        nms_ref[...] = ms

    @pl.when(i > 0)
    def _():
        nmd_ref[...] = jnp.maximum(nmd_ref[...], md)
        nms_ref[...] = jnp.maximum(nms_ref[...], ms)


def _tc_node1(x, W1, A_dst64, A_src64, A_dst16, A_src16):
    BN = 1000
    return pl.pallas_call(
        _node1_body,
        grid=(N // BN,),
        in_specs=[
            pl.BlockSpec((BN, D), lambda i: (i, 0)),
            pl.BlockSpec((D, F1), lambda i: (0, 0)),
            pl.BlockSpec((F1, F1), lambda i: (0, 0)),
            pl.BlockSpec((F1, F1), lambda i: (0, 0)),
            pl.BlockSpec((F1, 16), lambda i: (0, 0)),
            pl.BlockSpec((F1, 16), lambda i: (0, 0)),
        ],
        out_specs=[
            pl.BlockSpec((BN, 144), lambda i: (i, 0)),
            pl.BlockSpec((BN, 80), lambda i: (i, 0)),
            pl.BlockSpec((8, 16), lambda i: (0, 0)),
            pl.BlockSpec((8, 16), lambda i: (0, 0)),
        ],
        out_shape=[
            jax.ShapeDtypeStruct((N, 144), _f32),
            jax.ShapeDtypeStruct((N, 80), _f32),
            jax.ShapeDtypeStruct((8, 16), _f32),
            jax.ShapeDtypeStruct((8, 16), _f32),
        ],
    )(x, W1, A_dst64, A_src64, A_dst16, A_src16)


def _edge_body(ea_ref, w1_ref, w2_ref, o1_ref, o2_ref, em1_ref, em2_ref):
    i = pl.program_id(0)
    ea = ea_ref[...]
    rows = lax.broadcasted_iota(_i32, (ea.shape[0], 1), 0) + i * ea.shape[0]
    pad = rows >= (E // 32)
    o1 = jnp.dot(ea, w1_ref[...], preferred_element_type=_f32)
    o1 = jnp.where(pad, -1e30, o1)
    o1_ref[...] = o1
    o2 = jnp.dot(ea, w2_ref[...], preferred_element_type=_f32)
    o2 = jnp.where(pad, -1e30, o2)
    o2_ref[...] = o2
    e1 = jnp.broadcast_to(jnp.max(o1, axis=0, keepdims=True), (8, 2560))
    e2 = jnp.broadcast_to(jnp.max(o2, axis=0, keepdims=True), (8, 512))

    @pl.when(i == 0)
    def _():
        em1_ref[...] = e1
        em2_ref[...] = e2

    @pl.when(i > 0)
    def _():
        em1_ref[...] = jnp.maximum(em1_ref[...], e1)
        em2_ref[...] = jnp.maximum(em2_ref[...], e2)


def _tc_edge(ea_r, W_big1, W_big2):
    BR = 640
    return pl.pallas_call(
        _edge_body,
        grid=(EP // 32 // BR,),
        in_specs=[
            pl.BlockSpec((BR, 128), lambda i: (i, 0)),
            pl.BlockSpec((128, 2560), lambda i: (0, 0)),
            pl.BlockSpec((128, 512), lambda i: (0, 0)),
        ],
        out_specs=[
            pl.BlockSpec((BR, 2560), lambda i: (i, 0)),
            pl.BlockSpec((BR, 512), lambda i: (i, 0)),
            pl.BlockSpec((8, 2560), lambda i: (0, 0)),
            pl.BlockSpec((8, 512), lambda i: (0, 0)),
        ],
        out_shape=[
            jax.ShapeDtypeStruct((EP // 32, 2560), _f32),
            jax.ShapeDtypeStruct((EP // 32, 512), _f32),
            jax.ShapeDtypeStruct((8, 2560), _f32),
            jax.ShapeDtypeStruct((8, 512), _f32),
        ],
    )(ea_r, W_big1, W_big2)


def _mid_body(o0_ref, o1_ref, d0_ref, d1_ref, rm_ref, b1_ref, w2_ref,
              bd_ref, bs_ref, t_ref, td_ref, nmd_ref, nms_ref):
    i = pl.program_id(0)
    den64 = jnp.dot(d0_ref[...] + d1_ref[...], rm_ref[...],
                    preferred_element_type=_f32)
    h1 = (o0_ref[...] + o1_ref[...]) / (den64 + 1e-16) + b1_ref[...]
    hm = jnp.where(h1 > 0, h1, jnp.exp(h1) - 1.0)
    h2 = jnp.dot(hm, w2_ref[...], preferred_element_type=_f32)
    ad = jnp.dot(h2, bd_ref[...], preferred_element_type=_f32)
    td_ref[...] = ad
    asr = jnp.dot(h2, bs_ref[...], preferred_element_type=_f32)
    t_ref[...] = jnp.concatenate([asr, h2], axis=1)
    md = jnp.broadcast_to(jnp.max(ad, axis=0, keepdims=True), (8, 16))
    ms = jnp.broadcast_to(jnp.max(asr, axis=0, keepdims=True), (8, 16))

    @pl.when(i == 0)
    def _():
        nmd_ref[...] = md
        nms_ref[...] = ms

    @pl.when(i > 0)
    def _():
        nmd_ref[...] = jnp.maximum(nmd_ref[...], md)
        nms_ref[...] = jnp.maximum(nms_ref[...], ms)


def _tc_mid(o0, o1, d0, d1, Rm, b1, W2, B_d, B_s):
    BN = 1000
    return pl.pallas_call(
        _mid_body,
        grid=(N // BN,),
        in_specs=[
            pl.BlockSpec((BN, F1), lambda i: (i, 0)),
            pl.BlockSpec((BN, F1), lambda i: (i, 0)),
            pl.BlockSpec((BN, 16), lambda i: (i, 0)),
            pl.BlockSpec((BN, 16), lambda i: (i, 0)),
            pl.BlockSpec((16, F1), lambda i: (0, 0)),
            pl.BlockSpec((1, F1), lambda i: (0, 0)),
            pl.BlockSpec((F1, 16), lambda i: (0, 0)),
            pl.BlockSpec((16, 16), lambda i: (0, 0)),
            pl.BlockSpec((16, 16), lambda i: (0, 0)),
        ],
        out_specs=[
            pl.BlockSpec((BN, 2 * C), lambda i: (i, 0)),
            pl.BlockSpec((BN, C), lambda i: (i, 0)),
            pl.BlockSpec((8, C), lambda i: (0, 0)),
            pl.BlockSpec((8, C), lambda i: (0, 0)),
        ],
        out_shape=[
            jax.ShapeDtypeStruct((N, 2 * C), _f32),
            jax.ShapeDtypeStruct((N, C), _f32),
            jax.ShapeDtypeStruct((8, C), _f32),
            jax.ShapeDtypeStruct((8, C), _f32),
        ],
    )(o0, o1, d0, d1, Rm, b1, W2, B_d, B_s)


def _post_body(o0_ref, o1_ref, d0_ref, d1_ref, b2_ref, out_ref):
    o = (o0_ref[...] + o1_ref[...]) / (d0_ref[...] + d1_ref[...] + 1e-16)
    o = o + b2_ref[...]
    mx = jnp.max(o, axis=1, keepdims=True)
    s = o - mx
    ls = jnp.log(jnp.sum(jnp.exp(s), axis=1, keepdims=True))
    out_ref[...] = s - ls


def _tc_post(o0, o1, d0, d1, b2):
    BN = 1000
    return pl.pallas_call(
        _post_body,
        grid=(N // BN,),
        in_specs=[
            pl.BlockSpec((BN, C), lambda i: (i, 0)),
            pl.BlockSpec((BN, C), lambda i: (i, 0)),
            pl.BlockSpec((BN, C), lambda i: (i, 0)),
            pl.BlockSpec((BN, C), lambda i: (i, 0)),
            pl.BlockSpec((1, C), lambda i: (0, 0)),
        ],
        out_specs=pl.BlockSpec((BN, C), lambda i: (i, 0)),
        out_shape=jax.ShapeDtypeStruct((N, C), _f32),
    )(o0, o1, d0, d1, b2)


def _sc_layer(srcp, dstp, tdst, tsrc, ae, m, z, z16, Fdim):
    mesh = plsc.VectorSubcoreMesh(core_axis_name="c", subcore_axis_name="s",
                                  num_cores=2, num_subcores=16)
    NQ = Fdim // 16
    XW = 16 if Fdim > 16 else 0
    TW = 2 * Fdim + XW
    DW = Fdim + XW

    def body(src_hbm, dst_hbm, tdst_hbm, tsrc_hbm, ae_hbm, m_hbm,
             z_hbm, z16_hbm, outp_hbm, denp_hbm,
             idxs_v, idxd0, idxd1, ts_v, td_v, ae_v,
             ex0, ex1, hs0, hs1, m_v,
             out_sh, den_sh, sem_g, sem_s):
        idxd = [idxd0, idxd1]
        exv = [ex0, ex1]
        hsv = [hs0, hs1]

        ci = lax.axis_index("c")
        si = lax.axis_index("s")
        tid = ci * 16 + si
        r0 = si * ROWS_PER_SUB
        pltpu.sync_copy(z_hbm.at[pl.ds(r0, ROWS_PER_SUB)],
                        out_sh.at[pl.ds(r0, ROWS_PER_SUB)])
        pltpu.sync_copy(z16_hbm.at[pl.ds(r0, ROWS_PER_SUB)],
                        den_sh.at[pl.ds(r0, ROWS_PER_SUB)])
        pltpu.sync_copy(m_hbm, m_v)
        plsc.subcore_barrier()

        base_e = tid * PER_TILE
        mvecs = [m_v[pl.ds(cq * 16, 16)] for cq in range(DW // 16)]

        def gathers(it, s):
            off = base_e + it * EB
            pltpu.sync_copy(src_hbm.at[pl.ds(off, EB)], idxs_v)
            pltpu.sync_copy(dst_hbm.at[pl.ds(off, EB)], idxd[s])
            d1 = pltpu.make_async_copy(tsrc_hbm.at[idxs_v], ts_v, sem_g)
            d1.start()
            d2 = pltpu.make_async_copy(tdst_hbm.at[idxd[s]], td_v, sem_g)
            d2.start()
            d3 = pltpu.make_async_copy(ae_hbm.at[pl.ds(off, EB)], ae_v,
                                       sem_g)
            d3.start()
            d1.wait()
            d2.wait()
            d3.wait()

        def compute(s):
            def exrow(r, cc):
                e0 = None
                for cq in range(NQ):
                    sl = pl.ds(cq * 16, 16)
                    sv = ts_v[r, sl] + td_v[r, sl] + ae_v[r, sl]
                    sv = jnp.maximum(sv, 0.2 * sv) - mvecs[cq]
                    e = jnp.exp(sv)
                    if cq == 0:
                        e0 = e
                    hsv[s][r, sl] = ts_v[r, pl.ds(Fdim + cq * 16, 16)] * e
                if XW:
                    sv = (ts_v[r, pl.ds(2 * Fdim, 16)]
                          + td_v[r, pl.ds(Fdim, 16)]
                          + ae_v[r, pl.ds(Fdim, 16)])
                    sv = jnp.maximum(sv, 0.2 * sv) - mvecs[NQ]
                    exv[s][r] = jnp.exp(sv)
                else:
                    exv[s][r] = e0
                return cc

            lax.fori_loop(0, EB, exrow, 0)

        def scatters(s):
            d5 = pltpu.make_async_copy(exv[s], den_sh.at[idxd[s]], sem_s)
            d5.start(add=True)
            d6 = pltpu.make_async_copy(hsv[s], out_sh.at[idxd[s]], sem_s)
            d6.start(add=True)
            return d5, d6

        def chunk_pair(it2, carry):
            gathers(it2 * 2, 0)
            compute(0)
            dsa = scatters(0)
            gathers(it2 * 2 + 1, 1)
            compute(1)
            dsb = scatters(1)
            for d in dsa + dsb:
                d.wait()
            return carry

        lax.fori_loop(0, NCHUNK // 2, chunk_pair, 0)
        plsc.subcore_barrier()
        pltpu.sync_copy(out_sh.at[pl.ds(r0, ROWS_PER_SUB)],
                        outp_hbm.at[ci, pl.ds(r0, ROWS_PER_SUB)])
        pltpu.sync_copy(den_sh.at[pl.ds(r0, ROWS_PER_SUB)],
                        denp_hbm.at[ci, pl.ds(r0, ROWS_PER_SUB)])

    fn = pl.kernel(
        body,
        out_type=(jax.ShapeDtypeStruct((2, NP, Fdim), _f32),
                  jax.ShapeDtypeStruct((2, NP, 16), _f32)),
        mesh=mesh,
        scratch_types=[
            pltpu.VMEM((EB,), _i32),
            pltpu.VMEM((EB,), _i32),
            pltpu.VMEM((EB,), _i32),
            pltpu.VMEM((EB, TW), _f32),
            pltpu.VMEM((EB, DW), _f32),
            pltpu.VMEM((EB, DW), _f32),
            pltpu.VMEM((EB, 16), _f32),
            pltpu.VMEM((EB, 16), _f32),
            pltpu.VMEM((EB, Fdim), _f32),
            pltpu.VMEM((EB, Fdim), _f32),
            pltpu.VMEM((DW,), _f32),
            pltpu.VMEM_SHARED((NP, Fdim), _f32),
            pltpu.VMEM_SHARED((NP, 16), _f32),
            pltpu.SemaphoreType.DMA,
            pltpu.SemaphoreType.DMA,
        ],
        compiler_params=pltpu.CompilerParams(use_tc_tiling_on_sc=False),
    )
    return fn(srcp, dstp, tdst, tsrc, ae, m, z, z16)


def kernel(x, edge_index, edge_attr, W1, We1, att1, b1, W2, We2, att2, b2):
    src = edge_index[0].astype(_i32)
    dst = edge_index[1].astype(_i32)
    pad = EP - E
    srcp = jnp.concatenate([src, jnp.zeros((pad,), _i32)])
    dstp = jnp.concatenate([dst, jnp.zeros((pad,), _i32)])

    eye8 = jnp.eye(8, dtype=_f32)
    A_dst8 = jnp.einsum("ho,hg->hog", att1[:, :OUT], eye8).reshape(F1, 8)
    A_src8 = jnp.einsum("ho,hg->hog", att1[:, OUT:2 * OUT],
                        eye8).reshape(F1, 8)
    A_dst64 = jnp.repeat(A_dst8, 8, axis=1)
    A_src64 = jnp.repeat(A_src8, 8, axis=1)
    A_dst16 = jnp.tile(A_dst8, (1, 2))
    A_src16 = jnp.tile(A_src8, (1, 2))

    w_e1 = jnp.einsum("dho,ho->dh", We1.reshape(4, H, OUT), att1[:, 2 * OUT:])
    w_e1_80 = jnp.concatenate(
        [jnp.repeat(w_e1, 8, axis=1), jnp.tile(w_e1, (1, 2))], axis=1)
    w_e2 = We2 @ att2[0, 2 * C:]
    w_e2_16 = jnp.broadcast_to(w_e2[:, None], (4, 16))
    W_big1 = jnp.kron(jnp.eye(32, dtype=_f32), w_e1_80)
    W_big2 = jnp.kron(jnp.eye(32, dtype=_f32), w_e2_16)

    ea_pad = jnp.pad(edge_attr, ((0, pad), (0, 0)))
    ea_r = ea_pad.reshape(EP // 32, 128)

    B_d = jnp.tile(att2[0, :C][:, None], (1, 16))
    B_s = jnp.tile(att2[0, C:2 * C][:, None], (1, 16))
    Rm = jnp.concatenate(
        [jnp.kron(eye8, jnp.ones((1, 8), _f32)), jnp.zeros((8, F1), _f32)],
        axis=0)

    zf1 = jnp.zeros((NP, F1), _f32)
    z16 = jnp.zeros((NP, 16), _f32)

    ae1_r, ae2_r, em1, em2 = _tc_edge(ea_r, W_big1, W_big2)
    ae1 = ae1_r.reshape(EP, 80)
    ae2 = ae2_r.reshape(EP, C)
    em1_f = jnp.max(em1[0].reshape(32, 80), axis=0)
    em2_f = jnp.max(em2[0].reshape(32, C), axis=0)

    tsrc1, tdst1, nmd1, nms1 = _tc_node1(
        x, W1, A_dst64, A_src64, A_dst16, A_src16)
    m1_16 = nmd1[0] + nms1[0] + em1_f[F1:]
    m1 = jnp.concatenate([jnp.repeat(m1_16[:8], 8), m1_16])

    outp1, denp1 = _sc_layer(srcp, dstp, tdst1, tsrc1, ae1, m1,
                             zf1, z16, F1)

    tsrc2, tdst2, nmd2, nms2 = _tc_mid(
        outp1[0, :N], outp1[1, :N], denp1[0, :N], denp1[1, :N],
        Rm, b1.reshape(1, F1), W2, B_d, B_s)
    m2 = nmd2[0] + nms2[0] + em2_f

    outp2, denp2 = _sc_layer(srcp, dstp, tdst2, tsrc2, ae2, m2,
                             z16, z16, C)

    return _tc_post(outp2[0, :N], outp2[1, :N], denp2[0, :N], denp2[1, :N],
                    b2.reshape(1, C))


# speedup vs baseline: 2.0290x; 2.0290x over previous
import jax
import jax.numpy as jnp
from jax import lax
from jax.experimental import pallas as pl
from jax.experimental.pallas import tpu as pltpu
from jax.experimental.pallas import tpu_sc as plsc

N = 10000
E = 320000
D = 128
H = 8
OUT = 8
F1 = 64
C = 16

NTILES = 32
EB = 128
NCHUNK = 79
EP = NTILES * NCHUNK * EB
PER_TILE = EP // NTILES
NP = 10112
ROWS_PER_SUB = NP // 16

_f32 = jnp.float32
_i32 = jnp.int32


def _node1_body(x_ref, w_ref, ad_ref, as_ref, h_ref, a_d_ref, a_s_ref,
                nmd_ref, nms_ref):
    i = pl.program_id(0)
    h = jnp.dot(x_ref[...], w_ref[...], preferred_element_type=_f32)
    h_ref[...] = h
    ad = jnp.dot(h, ad_ref[...], preferred_element_type=_f32)
    a_d_ref[...] = ad
    asr = jnp.dot(h, as_ref[...], preferred_element_type=_f32)
    a_s_ref[...] = asr
    md = jnp.broadcast_to(jnp.max(ad, axis=0, keepdims=True), (8, F1))
    ms = jnp.broadcast_to(jnp.max(asr, axis=0, keepdims=True), (8, F1))

    @pl.when(i == 0)
    def _():
        nmd_ref[...] = md
        nms_ref[...] = ms

    @pl.when(i > 0)
    def _():
        nmd_ref[...] = jnp.maximum(nmd_ref[...], md)
        nms_ref[...] = jnp.maximum(nms_ref[...], ms)


def _tc_node1(x, W1, A_dst, A_src):
    BN = 1000
    return pl.pallas_call(
        _node1_body,
        grid=(N // BN,),
        in_specs=[
            pl.BlockSpec((BN, D), lambda i: (i, 0)),
            pl.BlockSpec((D, F1), lambda i: (0, 0)),
            pl.BlockSpec((F1, F1), lambda i: (0, 0)),
            pl.BlockSpec((F1, F1), lambda i: (0, 0)),
        ],
        out_specs=[
            pl.BlockSpec((BN, F1), lambda i: (i, 0)),
            pl.BlockSpec((BN, F1), lambda i: (i, 0)),
            pl.BlockSpec((BN, F1), lambda i: (i, 0)),
            pl.BlockSpec((8, F1), lambda i: (0, 0)),
            pl.BlockSpec((8, F1), lambda i: (0, 0)),
        ],
        out_shape=[
            jax.ShapeDtypeStruct((N, F1), _f32),
            jax.ShapeDtypeStruct((N, F1), _f32),
            jax.ShapeDtypeStruct((N, F1), _f32),
            jax.ShapeDtypeStruct((8, F1), _f32),
            jax.ShapeDtypeStruct((8, F1), _f32),
        ],
    )(x, W1, A_dst, A_src)


def _edge_body(ea_ref, w1_ref, w2_ref, o1_ref, o2_ref, em1_ref, em2_ref):
    i = pl.program_id(0)
    ea = ea_ref[...]
    rows = lax.broadcasted_iota(_i32, (ea.shape[0], 1), 0) + i * ea.shape[0]
    pad = rows >= (E // 32)
    o1 = jnp.dot(ea, w1_ref[...], preferred_element_type=_f32)
    o1 = jnp.where(pad, -1e30, o1)
    o1_ref[...] = o1
    o2 = jnp.dot(ea, w2_ref[...], preferred_element_type=_f32)
    o2 = jnp.where(pad, -1e30, o2)
    o2_ref[...] = o2
    e1 = jnp.broadcast_to(jnp.max(o1, axis=0, keepdims=True), (8, 2048))
    e2 = jnp.broadcast_to(jnp.max(o2, axis=0, keepdims=True), (8, 512))

    @pl.when(i == 0)
    def _():
        em1_ref[...] = e1
        em2_ref[...] = e2

    @pl.when(i > 0)
    def _():
        em1_ref[...] = jnp.maximum(em1_ref[...], e1)
        em2_ref[...] = jnp.maximum(em2_ref[...], e2)


def _tc_edge(ea_r, W_big1, W_big2):
    BR = 632
    return pl.pallas_call(
        _edge_body,
        grid=(EP // 32 // BR,),
        in_specs=[
            pl.BlockSpec((BR, 128), lambda i: (i, 0)),
            pl.BlockSpec((128, 2048), lambda i: (0, 0)),
            pl.BlockSpec((128, 512), lambda i: (0, 0)),
        ],
        out_specs=[
            pl.BlockSpec((BR, 2048), lambda i: (i, 0)),
            pl.BlockSpec((BR, 512), lambda i: (i, 0)),
            pl.BlockSpec((8, 2048), lambda i: (0, 0)),
            pl.BlockSpec((8, 512), lambda i: (0, 0)),
        ],
        out_shape=[
            jax.ShapeDtypeStruct((EP // 32, 2048), _f32),
            jax.ShapeDtypeStruct((EP // 32, 512), _f32),
            jax.ShapeDtypeStruct((8, 2048), _f32),
            jax.ShapeDtypeStruct((8, 512), _f32),
        ],
    )(ea_r, W_big1, W_big2)


def _mid_body(o0_ref, o1_ref, d0_ref, d1_ref, b1_ref, w2_ref,
              bd_ref, bs_ref, h2_ref, a_d_ref, a_s_ref, nmd_ref, nms_ref):
    i = pl.program_id(0)
    h1 = ((o0_ref[...] + o1_ref[...])
          / (d0_ref[...] + d1_ref[...] + 1e-16) + b1_ref[...])
    hm = jnp.where(h1 > 0, h1, jnp.exp(h1) - 1.0)
    h2 = jnp.dot(hm, w2_ref[...], preferred_element_type=_f32)
    h2_ref[...] = h2
    ad = jnp.dot(h2, bd_ref[...], preferred_element_type=_f32)
    a_d_ref[...] = ad
    asr = jnp.dot(h2, bs_ref[...], preferred_element_type=_f32)
    a_s_ref[...] = asr
    md = jnp.broadcast_to(jnp.max(ad, axis=0, keepdims=True), (8, 16))
    ms = jnp.broadcast_to(jnp.max(asr, axis=0, keepdims=True), (8, 16))

    @pl.when(i == 0)
    def _():
        nmd_ref[...] = md
        nms_ref[...] = ms

    @pl.when(i > 0)
    def _():
        nmd_ref[...] = jnp.maximum(nmd_ref[...], md)
        nms_ref[...] = jnp.maximum(nms_ref[...], ms)


def _tc_mid(o0, o1, d0, d1, b1, W2, B_d, B_s):
    BN = 1000
    return pl.pallas_call(
        _mid_body,
        grid=(N // BN,),
        in_specs=[
            pl.BlockSpec((BN, F1), lambda i: (i, 0)),
            pl.BlockSpec((BN, F1), lambda i: (i, 0)),
            pl.BlockSpec((BN, F1), lambda i: (i, 0)),
            pl.BlockSpec((BN, F1), lambda i: (i, 0)),
            pl.BlockSpec((1, F1), lambda i: (0, 0)),
            pl.BlockSpec((F1, 16), lambda i: (0, 0)),
            pl.BlockSpec((16, 16), lambda i: (0, 0)),
            pl.BlockSpec((16, 16), lambda i: (0, 0)),
        ],
        out_specs=[
            pl.BlockSpec((BN, 16), lambda i: (i, 0)),
            pl.BlockSpec((BN, 16), lambda i: (i, 0)),
            pl.BlockSpec((BN, 16), lambda i: (i, 0)),
            pl.BlockSpec((8, 16), lambda i: (0, 0)),
            pl.BlockSpec((8, 16), lambda i: (0, 0)),
        ],
        out_shape=[
            jax.ShapeDtypeStruct((N, 16), _f32),
            jax.ShapeDtypeStruct((N, 16), _f32),
            jax.ShapeDtypeStruct((N, 16), _f32),
            jax.ShapeDtypeStruct((8, 16), _f32),
            jax.ShapeDtypeStruct((8, 16), _f32),
        ],
    )(o0, o1, d0, d1, b1, W2, B_d, B_s)


def _post_body(o0_ref, o1_ref, d0_ref, d1_ref, b2_ref, out_ref):
    o = (o0_ref[...] + o1_ref[...]) / (d0_ref[...] + d1_ref[...] + 1e-16)
    o = o + b2_ref[...]
    mx = jnp.max(o, axis=1, keepdims=True)
    s = o - mx
    ls = jnp.log(jnp.sum(jnp.exp(s), axis=1, keepdims=True))
    out_ref[...] = s - ls


def _tc_post(o0, o1, d0, d1, b2):
    BN = 1000
    return pl.pallas_call(
        _post_body,
        grid=(N // BN,),
        in_specs=[
            pl.BlockSpec((BN, C), lambda i: (i, 0)),
            pl.BlockSpec((BN, C), lambda i: (i, 0)),
            pl.BlockSpec((BN, C), lambda i: (i, 0)),
            pl.BlockSpec((BN, C), lambda i: (i, 0)),
            pl.BlockSpec((1, C), lambda i: (0, 0)),
        ],
        out_specs=pl.BlockSpec((BN, C), lambda i: (i, 0)),
        out_shape=jax.ShapeDtypeStruct((N, C), _f32),
    )(o0, o1, d0, d1, b2)


def _sc_layer(srcp, dstp, adst, asrc, ae, hT, m, z, Fdim):
    mesh = plsc.VectorSubcoreMesh(core_axis_name="c", subcore_axis_name="s",
                                  num_cores=2, num_subcores=16)
    NQ = Fdim // 16

    def body(src_hbm, dst_hbm, adst_hbm, asrc_hbm, ae_hbm, h_hbm, m_hbm,
             z_hbm, outp_hbm, denp_hbm,
             idxs_v, idxd_v, as_v, ad_v, ae_v, ex_v, h_v, m_v,
             out_sh, den_sh, sem):
        ci = lax.axis_index("c")
        si = lax.axis_index("s")
        tid = ci * 16 + si
        r0 = si * ROWS_PER_SUB
        pltpu.sync_copy(z_hbm.at[pl.ds(r0, ROWS_PER_SUB)],
                        out_sh.at[pl.ds(r0, ROWS_PER_SUB)])
        pltpu.sync_copy(z_hbm.at[pl.ds(r0, ROWS_PER_SUB)],
                        den_sh.at[pl.ds(r0, ROWS_PER_SUB)])
        pltpu.sync_copy(m_hbm, m_v)
        plsc.subcore_barrier()

        base_e = tid * PER_TILE
        mvecs = [m_v[pl.ds(cq * 16, 16)] for cq in range(NQ)]

        def chunk(it, carry):
            off = base_e + it * EB
            pltpu.sync_copy(src_hbm.at[pl.ds(off, EB)], idxs_v)
            pltpu.sync_copy(dst_hbm.at[pl.ds(off, EB)], idxd_v)
            d1 = pltpu.make_async_copy(asrc_hbm.at[idxs_v], as_v, sem)
            d1.start()
            d2 = pltpu.make_async_copy(adst_hbm.at[idxd_v], ad_v, sem)
            d2.start()
            d3 = pltpu.make_async_copy(h_hbm.at[idxs_v], h_v, sem)
            d3.start()
            d4 = pltpu.make_async_copy(ae_hbm.at[pl.ds(off, EB)], ae_v, sem)
            d4.start()
            d1.wait()
            d2.wait()
            d3.wait()
            d4.wait()

            def exrow(r, cc):
                for cq in range(NQ):
                    sl = pl.ds(cq * 16, 16)
                    s = as_v[r, sl] + ad_v[r, sl] + ae_v[r, sl]
                    s = jnp.maximum(s, 0.2 * s) - mvecs[cq]
                    e = jnp.exp(s)
                    ex_v[r, sl] = e
                    h_v[r, sl] = h_v[r, sl] * e
                return cc

            lax.fori_loop(0, EB, exrow, 0)
            pltpu.sync_copy(ex_v, den_sh.at[idxd_v], add=True)
            pltpu.sync_copy(h_v, out_sh.at[idxd_v], add=True)
            return carry

        lax.fori_loop(0, NCHUNK, chunk, 0)
        plsc.subcore_barrier()
        pltpu.sync_copy(out_sh.at[pl.ds(r0, ROWS_PER_SUB)],
                        outp_hbm.at[ci, pl.ds(r0, ROWS_PER_SUB)])
        pltpu.sync_copy(den_sh.at[pl.ds(r0, ROWS_PER_SUB)],
                        denp_hbm.at[ci, pl.ds(r0, ROWS_PER_SUB)])

    fn = pl.kernel(
        body,
        out_type=(jax.ShapeDtypeStruct((2, NP, Fdim), _f32),
                  jax.ShapeDtypeStruct((2, NP, Fdim), _f32)),
        mesh=mesh,
        scratch_types=[
            pltpu.VMEM((EB,), _i32),
            pltpu.VMEM((EB,), _i32),
            pltpu.VMEM((EB, Fdim), _f32),
            pltpu.VMEM((EB, Fdim), _f32),
            pltpu.VMEM((EB, Fdim), _f32),
            pltpu.VMEM((EB, Fdim), _f32),
            pltpu.VMEM((EB, Fdim), _f32),
            pltpu.VMEM((Fdim,), _f32),
            pltpu.VMEM_SHARED((NP, Fdim), _f32),
            pltpu.VMEM_SHARED((NP, Fdim), _f32),
            pltpu.SemaphoreType.DMA,
        ],
        compiler_params=pltpu.CompilerParams(use_tc_tiling_on_sc=False),
    )
    return fn(srcp, dstp, adst, asrc, ae, hT, m, z)


def kernel(x, edge_index, edge_attr, W1, We1, att1, b1, W2, We2, att2, b2):
    src = edge_index[0].astype(_i32)
    dst = edge_index[1].astype(_i32)
    pad = EP - E
    srcp = jnp.concatenate([src, jnp.zeros((pad,), _i32)])
    dstp = jnp.concatenate([dst, jnp.zeros((pad,), _i32)])

    eye8 = jnp.eye(8, dtype=_f32)
    A_dst = jnp.repeat(
        jnp.einsum("ho,hg->hog", att1[:, :OUT], eye8).reshape(F1, 8),
        8, axis=1)
    A_src = jnp.repeat(
        jnp.einsum("ho,hg->hog", att1[:, OUT:2 * OUT], eye8).reshape(F1, 8),
        8, axis=1)

    w_e1 = jnp.einsum("dho,ho->dh", We1.reshape(4, H, OUT), att1[:, 2 * OUT:])
    w_e1_64 = jnp.repeat(w_e1, 8, axis=1)
    w_e2 = We2 @ att2[0, 2 * C:]
    w_e2_16 = jnp.broadcast_to(w_e2[:, None], (4, 16))
    W_big1 = jnp.kron(jnp.eye(32, dtype=_f32), w_e1_64)
    W_big2 = jnp.kron(jnp.eye(32, dtype=_f32), w_e2_16)

    ea_pad = jnp.pad(edge_attr, ((0, pad), (0, 0)))
    ea_r = ea_pad.reshape(EP // 32, 128)

    B_d = jnp.tile(att2[0, :C][:, None], (1, 16))
    B_s = jnp.tile(att2[0, C:2 * C][:, None], (1, 16))

    zf1 = jnp.zeros((NP, F1), _f32)
    zf2 = jnp.zeros((NP, C), _f32)

    ae1_r, ae2_r, em1, em2 = _tc_edge(ea_r, W_big1, W_big2)
    ae1 = ae1_r.reshape(EP, F1)
    ae2 = ae2_r.reshape(EP, C)
    em1_f = jnp.max(em1[0].reshape(32, F1), axis=0)
    em2_f = jnp.max(em2[0].reshape(32, C), axis=0)

    h1, adst1, asrc1, nmd1, nms1 = _tc_node1(x, W1, A_dst, A_src)
    m1 = nmd1[0] + nms1[0] + em1_f

    outp1, denp1 = _sc_layer(srcp, dstp, adst1, asrc1, ae1, h1, m1, zf1, F1)

    h2, adst2, asrc2, nmd2, nms2 = _tc_mid(
        outp1[0, :N], outp1[1, :N], denp1[0, :N], denp1[1, :N],
        b1.reshape(1, F1), W2, B_d, B_s)
    m2 = nmd2[0] + nms2[0] + em2_f

    outp2, denp2 = _sc_layer(srcp, dstp, adst2, asrc2, ae2, h2, m2, zf2, C)

    return _tc_post(outp2[0, :N], outp2[1, :N], denp2[0, :N], denp2[1, :N],
                    b2.reshape(1, C))
